# Initial kernel scaffold; baseline (speedup 1.0000x reference)
import jax
import jax.numpy as jnp
from jax.experimental import pallas as pl


def kernel(h, adj, W, Wl, Wr, b):
    raise NotImplementedError("write your pallas kernel here")


# speedup vs baseline: 1.3512x; 1.3512x over previous
import functools

import jax
import jax.numpy as jnp
from jax.experimental import pallas as pl
from jax.experimental.pallas import tpu as pltpu

N = 8192
D = 128

BR = 256
BC = 2048


def _proj_kernel(h_ref, w_ref, wl_ref, wrt_ref, x_ref, ea_ref, ec_ref,
                 eb_ref, ed_ref):
    x = jnp.dot(h_ref[...], w_ref[...], preferred_element_type=jnp.float32)
    x_ref[...] = x
    el = jnp.dot(x, wl_ref[...], preferred_element_type=jnp.float32)
    ea_ref[...] = jnp.exp(el)
    ec_ref[...] = jnp.exp(0.2 * el)
    er_t = jax.lax.dot_general(wrt_ref[...], x,
                               dimension_numbers=(((1,), (1,)), ((), ())),
                               preferred_element_type=jnp.float32)
    eb_ref[...] = jnp.exp(er_t)
    ed_ref[...] = jnp.exp(0.2 * er_t)


def _gat_kernel(adj_ref, x_ref, ea_ref, ec_ref, eb_ref, ed_ref, b_ref,
                out_ref, acc_ref, den_ref):
    c = pl.program_id(1)
    nc = pl.num_programs(1)

    @pl.when(c == 0)
    def _():
        acc_ref[...] = jnp.zeros_like(acc_ref)
        den_ref[...] = jnp.zeros_like(den_ref)

    ea = ea_ref[...]
    ec = ec_ref[...]
    eb = eb_ref[...]
    ed = ed_ref[...]

    pos = ea * eb
    neg = ec * ed
    p = jnp.where(pos >= 1.0, pos, neg)
    a = jnp.where(adj_ref[...] > 0, p, 0.0)

    xcol = x_ref[pl.ds(c * BC, BC), :]
    acc_ref[...] += jnp.dot(a, xcol, preferred_element_type=jnp.float32)
    den_ref[...] += jnp.sum(a, axis=1, keepdims=True)

    @pl.when(c == nc - 1)
    def _():
        out_ref[...] = (acc_ref[...] / jnp.maximum(den_ref[...], 1e-12)
                        + b_ref[...])


@jax.jit
def kernel(h, adj, W, Wl, Wr, b):
    n, d = h.shape

    pr = 512
    x, ea, ec, eb, ed = pl.pallas_call(
        _proj_kernel,
        grid=(n // pr,),
        in_specs=[
            pl.BlockSpec((pr, d), lambda r: (r, 0)),
            pl.BlockSpec((d, d), lambda r: (0, 0)),
            pl.BlockSpec((d, 1), lambda r: (0, 0)),
            pl.BlockSpec((1, d), lambda r: (0, 0)),
        ],
        out_specs=[
            pl.BlockSpec((pr, d), lambda r: (r, 0)),
            pl.BlockSpec((pr, 1), lambda r: (r, 0)),
            pl.BlockSpec((pr, 1), lambda r: (r, 0)),
            pl.BlockSpec((1, pr), lambda r: (0, r)),
            pl.BlockSpec((1, pr), lambda r: (0, r)),
        ],
        out_shape=[
            jax.ShapeDtypeStruct((n, d), jnp.float32),
            jax.ShapeDtypeStruct((n, 1), jnp.float32),
            jax.ShapeDtypeStruct((n, 1), jnp.float32),
            jax.ShapeDtypeStruct((1, n), jnp.float32),
            jax.ShapeDtypeStruct((1, n), jnp.float32),
        ],
    )(h, W, Wl, Wr.T)

    out = pl.pallas_call(
        _gat_kernel,
        grid=(n // BR, n // BC),
        in_specs=[
            pl.BlockSpec((BR, BC), lambda r, c: (r, c)),
            pl.BlockSpec((n, d), lambda r, c: (0, 0)),
            pl.BlockSpec((BR, 1), lambda r, c: (r, 0)),
            pl.BlockSpec((BR, 1), lambda r, c: (r, 0)),
            pl.BlockSpec((1, BC), lambda r, c: (0, c)),
            pl.BlockSpec((1, BC), lambda r, c: (0, c)),
            pl.BlockSpec((1, d), lambda r, c: (0, 0)),
        ],
        out_specs=pl.BlockSpec((BR, d), lambda r, c: (r, 0)),
        out_shape=jax.ShapeDtypeStruct((n, d), jnp.float32),
        scratch_shapes=[
            pltpu.VMEM((BR, d), jnp.float32),
            pltpu.VMEM((BR, 1), jnp.float32),
        ],
        compiler_params=pltpu.CompilerParams(
            dimension_semantics=("parallel", "arbitrary"),
        ),
    )(adj, x, ea, ec, eb, ed, b.reshape(1, d))
    return out
